# baseline (speedup 1.0000x reference)
import functools

import jax
import jax.numpy as jnp
from jax import lax
from jax.experimental import pallas as pl
from jax.experimental.pallas import tpu as pltpu
from jax.experimental.pallas import tpu_sc as plsc

VOCAB = 1000000
DIM = 64
B = 4096
L = 50

NC = 2
NS = 16
LANES = 16
NW = NC * NS

ROWS = B * L
RPW = ROWS // NW
CHUNK = 128
NCHUNK = RPW // CHUNK
NGRP = CHUNK // LANES
OPC = CHUNK // 2

_MAGIC = 0x5F3759DF


def _rsqrt(ssv):
    bits = plsc.bitcast(ssv, jnp.int32)
    y = plsc.bitcast(_MAGIC - (bits >> 1), jnp.float32)
    for _ in range(3):
        y = y * (1.5 - 0.5 * (ssv * y) * y)
    return y


_mesh = plsc.VectorSubcoreMesh(core_axis_name="c", subcore_axis_name="s")


@functools.partial(
    pl.kernel,
    mesh=_mesh,
    out_type=jax.ShapeDtypeStruct((ROWS // 2, DIM * 2), jnp.float32),
    scratch_types=[
        pltpu.VMEM((NCHUNK, CHUNK), jnp.int32),
        pltpu.VMEM((NCHUNK, CHUNK), jnp.int32),
        pltpu.VMEM((NCHUNK, CHUNK), jnp.int32),
        pltpu.VMEM((CHUNK, DIM * 2), jnp.float32),
        pltpu.VMEM((OPC, DIM * 2), jnp.float32),
        pltpu.SemaphoreType.DMA,
        pltpu.SemaphoreType.DMA,
    ],
    compiler_params=pltpu.CompilerParams(needs_layout_passes=False),
)
def _sc_lookup(idx_hbm, half_hbm, mask_hbm, table_hbm, out_hbm,
               idx_v, half_v, msk_v, buf, obuf, gsem, osem):
    wid = lax.axis_index("s") * NC + lax.axis_index("c")

    pltpu.sync_copy(idx_hbm.at[wid], idx_v)
    pltpu.sync_copy(half_hbm.at[wid], half_v)
    pltpu.sync_copy(mask_hbm.at[wid], msk_v)

    def chunk_body(j, carry):
        pltpu.async_copy(table_hbm.at[idx_v.at[j]], buf, gsem).wait()

        def grp_body(g, c2):
            iota = lax.iota(jnp.int32, LANES)
            rows = g * LANES + iota
            h = half_v[j, pl.ds(g * LANES, LANES)]
            hcol = h * DIM
            ss = None
            for k in range(DIM):
                v = plsc.load_gather(buf, [rows, hcol + k])
                ss = v * v if ss is None else ss + v * v
            y = _rsqrt(ss)
            norm = ss * y
            mrow = msk_v[j, pl.ds(g * LANES, LANES)]
            bm = jnp.where(mrow > 0, 0.0, 1.0)
            inv = jnp.where(norm >= 1e-6, y, 1e6) * bm
            prow = g * (LANES // 2) + (iota >> 1)
            pcol = (iota & 1) * DIM
            for k in range(DIM):
                v = plsc.load_gather(buf, [rows, hcol + k])
                plsc.store_scatter(obuf, [prow, pcol + k], v * inv)
            return c2

        lax.fori_loop(0, NGRP, grp_body, 0)
        pltpu.async_copy(
            obuf, out_hbm.at[pl.ds(wid * (RPW // 2) + j * OPC, OPC)], osem
        ).wait()
        return carry

    lax.fori_loop(0, NCHUNK, chunk_body, 0)


def kernel(indices, mask, table):
    flat = indices.reshape(NW, NCHUNK, CHUNK)
    idx3 = flat >> 1
    half3 = flat & 1
    mask3 = mask.astype(jnp.int32).reshape(NW, NCHUNK, CHUNK)
    table2 = table.reshape(VOCAB // 2, DIM * 2)
    out = _sc_lookup(idx3, half3, mask3, table2)
    return out.reshape(B, L, DIM)


# speedup vs baseline: 1.5751x; 1.5751x over previous
import functools

import jax
import jax.numpy as jnp
from jax import lax
from jax.experimental import pallas as pl
from jax.experimental.pallas import tpu as pltpu
from jax.experimental.pallas import tpu_sc as plsc

VOCAB = 1000000
DIM = 64
B = 4096
L = 50

NC = 2
NS = 16
LANES = 16
NW = NC * NS

ROWS = B * L
RPW = ROWS // NW
CHUNK = 128
NCHUNK = RPW // CHUNK
NGRP = CHUNK // LANES
OPC = CHUNK // 2

TBLK = 4096
NTBLK = (VOCAB + TBLK - 1) // TBLK
PREC = TBLK // 2
PTAB = NTBLK * PREC

_MAGIC = 0x5F3759DF


def _pack_body(tt_ref, out_ref):
    xt = tt_ref[...].T
    out_ref[...] = jnp.concatenate([xt[:PREC], xt[PREC:]], axis=1)


_pack_table = pl.pallas_call(
    _pack_body,
    grid=(NTBLK,),
    in_specs=[pl.BlockSpec((DIM, TBLK), lambda c: (0, c))],
    out_specs=pl.BlockSpec((PREC, DIM * 2), lambda c: (c, 0)),
    out_shape=jax.ShapeDtypeStruct((PTAB, DIM * 2), jnp.float32),
)


def _rsqrt(ssv):
    bits = plsc.bitcast(ssv, jnp.int32)
    y = plsc.bitcast(_MAGIC - (bits >> 1), jnp.float32)
    for _ in range(3):
        y = y * (1.5 - 0.5 * (ssv * y) * y)
    return y


_mesh = plsc.VectorSubcoreMesh(core_axis_name="c", subcore_axis_name="s")


@functools.partial(
    pl.kernel,
    mesh=_mesh,
    out_type=jax.ShapeDtypeStruct((L, DIM, B), jnp.float32),
    scratch_types=[
        pltpu.VMEM((NCHUNK, CHUNK), jnp.int32),
        pltpu.VMEM((NCHUNK, CHUNK), jnp.int32),
        pltpu.VMEM((NCHUNK, CHUNK), jnp.int32),
        pltpu.VMEM((CHUNK, DIM * 2), jnp.float32),
        pltpu.VMEM((DIM, CHUNK), jnp.float32),
        pltpu.SemaphoreType.DMA,
        pltpu.SemaphoreType.DMA,
    ],
    compiler_params=pltpu.CompilerParams(needs_layout_passes=False),
)
def _sc_lookup(idx_hbm, half_hbm, mask_hbm, table_hbm, out_hbm,
               idx_v, half_v, msk_v, buf, obuf, gsem, osem):
    wid = lax.axis_index("s") * NC + lax.axis_index("c")

    pltpu.sync_copy(idx_hbm.at[wid], idx_v)
    pltpu.sync_copy(half_hbm.at[wid], half_v)
    pltpu.sync_copy(mask_hbm.at[wid], msk_v)

    def chunk_body(j, carry):
        pltpu.async_copy(table_hbm.at[idx_v.at[j]], buf, gsem).wait()

        def grp_body(g, c2):
            iota = lax.iota(jnp.int32, LANES)
            rows = g * LANES + iota
            zero_i = iota * 0
            h = half_v[j, pl.ds(g * LANES, LANES)]
            hcol = h * DIM
            ss = None
            for k in range(DIM):
                v = plsc.load_gather(buf, [rows, hcol + k])
                ss = v * v if ss is None else ss + v * v
            y = _rsqrt(ss)
            norm = ss * y
            mrow = msk_v[j, pl.ds(g * LANES, LANES)]
            bm = jnp.where(mrow > 0, 0.0, 1.0)
            inv = jnp.where(norm >= 1e-6, y, 1e6) * bm
            for k in range(DIM):
                v = plsc.load_gather(buf, [rows, hcol + k])
                plsc.store_scatter(obuf, [zero_i + k, rows], v * inv)
            return c2

        lax.fori_loop(0, NGRP, grp_body, 0)
        pltpu.async_copy(
            obuf, out_hbm.at[j, :, pl.ds(wid * CHUNK, CHUNK)], osem
        ).wait()
        return carry

    lax.fori_loop(0, NCHUNK, chunk_body, 0)


def kernel(indices, mask, table):
    slab = indices.reshape(NW, CHUNK, L).transpose(0, 2, 1)
    blk = slab >> 12
    q = slab & (TBLK - 1)
    half3 = q >> 11
    idx3 = (blk << 11) | (q & (PREC - 1))
    mask3 = mask.astype(jnp.int32).reshape(NW, CHUNK, L).transpose(0, 2, 1)
    table2 = _pack_table(table.T)
    out = _sc_lookup(idx3, half3, mask3, table2)
    return out.transpose(2, 0, 1)


# speedup vs baseline: 1.7340x; 1.1009x over previous
import functools

import jax
import jax.numpy as jnp
from jax import lax
from jax.experimental import pallas as pl
from jax.experimental.pallas import tpu as pltpu
from jax.experimental.pallas import tpu_sc as plsc

VOCAB = 1000000
DIM = 64
B = 4096
L = 50

NC = 2
NS = 16
LANES = 16
NW = NC * NS

ROWS = B * L
RPW = ROWS // NW
CHUNK = 128
NCHUNK = RPW // CHUNK
NGRP = CHUNK // LANES
OPC = CHUNK // 2

TBLK = 4096
NTBLK = (VOCAB + TBLK - 1) // TBLK
PREC = TBLK // 2
PTAB = NTBLK * PREC

_MAGIC = 0x5F3759DF


def _pack_body(tt_ref, out_ref):
    xt = tt_ref[...].T
    out_ref[...] = jnp.concatenate([xt[:PREC], xt[PREC:]], axis=1)


_pack_table = pl.pallas_call(
    _pack_body,
    grid=(NTBLK,),
    in_specs=[pl.BlockSpec((DIM, TBLK), lambda c: (0, c))],
    out_specs=pl.BlockSpec((PREC, DIM * 2), lambda c: (c, 0)),
    out_shape=jax.ShapeDtypeStruct((PTAB, DIM * 2), jnp.float32),
)


def _rsqrt(ssv):
    bits = plsc.bitcast(ssv, jnp.int32)
    y = plsc.bitcast(_MAGIC - (bits >> 1), jnp.float32)
    for _ in range(3):
        y = y * (1.5 - 0.5 * (ssv * y) * y)
    return y


_mesh = plsc.VectorSubcoreMesh(core_axis_name="c", subcore_axis_name="s")


@functools.partial(
    pl.kernel,
    mesh=_mesh,
    out_type=jax.ShapeDtypeStruct((L, DIM, B), jnp.float32),
    scratch_types=[
        pltpu.VMEM((NCHUNK, CHUNK), jnp.int32),
        pltpu.VMEM((NCHUNK, CHUNK), jnp.int32),
        pltpu.VMEM((NCHUNK, CHUNK), jnp.int32),
        pltpu.VMEM((CHUNK, DIM * 2), jnp.float32),
        pltpu.VMEM((CHUNK, DIM * 2), jnp.float32),
        pltpu.VMEM((DIM, CHUNK), jnp.float32),
        pltpu.VMEM((DIM, CHUNK), jnp.float32),
        pltpu.SemaphoreType.DMA,
        pltpu.SemaphoreType.DMA,
        pltpu.SemaphoreType.DMA,
        pltpu.SemaphoreType.DMA,
    ],
    compiler_params=pltpu.CompilerParams(needs_layout_passes=False),
)
def _sc_lookup(idx_hbm, half_hbm, mask_hbm, table_hbm, out_hbm,
               idx_v, half_v, msk_v, buf0, buf1, obuf0, obuf1,
               gsem0, gsem1, osem0, osem1):
    wid = lax.axis_index("s") * NC + lax.axis_index("c")
    col0 = wid * CHUNK

    pltpu.sync_copy(idx_hbm.at[wid], idx_v)
    pltpu.sync_copy(half_hbm.at[wid], half_v)
    pltpu.sync_copy(mask_hbm.at[wid], msk_v)

    def compute(j, buf, obuf):
        def grp_body(g, c2):
            iota = lax.iota(jnp.int32, LANES)
            rows = g * LANES + iota
            zero_i = iota * 0
            h = half_v[j, pl.ds(g * LANES, LANES)]
            hcol = h * DIM
            ss = None
            for k in range(DIM):
                v = plsc.load_gather(buf, [rows, hcol + k])
                ss = v * v if ss is None else ss + v * v
            y = _rsqrt(ss)
            norm = ss * y
            mrow = msk_v[j, pl.ds(g * LANES, LANES)]
            bm = jnp.where(mrow > 0, 0.0, 1.0)
            inv = jnp.where(norm >= 1e-6, y, 1e6) * bm
            for k in range(DIM):
                v = plsc.load_gather(buf, [rows, hcol + k])
                plsc.store_scatter(obuf, [zero_i + k, rows], v * inv)
            return c2

        lax.fori_loop(0, NGRP, grp_body, 0)

    def gather(j, buf, sem):
        return pltpu.async_copy(table_hbm.at[idx_v.at[j]], buf, sem)

    def writeback(j, obuf, sem):
        return pltpu.async_copy(obuf, out_hbm.at[j, :, pl.ds(col0, CHUNK)], sem)

    gather(0, buf0, gsem0)
    gather(1, buf1, gsem1)

    def stage(t, j, buf, obuf, gsem, osem):
        pltpu.make_async_copy(table_hbm.at[idx_v.at[j]], buf, gsem).wait()

        @pl.when(t > 0)
        def _():
            pltpu.make_async_copy(
                obuf, out_hbm.at[j, :, pl.ds(col0, CHUNK)], osem
            ).wait()

        compute(j, buf, obuf)
        writeback(j, obuf, osem)
        gather(jnp.minimum(j + 2, NCHUNK - 1), buf, gsem)

    def outer(t, carry):
        stage(t, 2 * t, buf0, obuf0, gsem0, osem0)
        stage(t, 2 * t + 1, buf1, obuf1, gsem1, osem1)
        return carry

    lax.fori_loop(0, NCHUNK // 2, outer, 0)

    pltpu.make_async_copy(table_hbm.at[idx_v.at[0]], buf0, gsem0).wait()
    pltpu.make_async_copy(table_hbm.at[idx_v.at[0]], buf1, gsem1).wait()
    pltpu.make_async_copy(obuf0, out_hbm.at[0, :, pl.ds(col0, CHUNK)], osem0).wait()
    pltpu.make_async_copy(obuf1, out_hbm.at[0, :, pl.ds(col0, CHUNK)], osem1).wait()


def kernel(indices, mask, table):
    slab = indices.reshape(NW, CHUNK, L).transpose(0, 2, 1)
    blk = slab >> 12
    q = slab & (TBLK - 1)
    half3 = q >> 11
    idx3 = (blk << 11) | (q & (PREC - 1))
    mask3 = mask.astype(jnp.int32).reshape(NW, CHUNK, L).transpose(0, 2, 1)
    table2 = _pack_table(table.T)
    out = _sc_lookup(idx3, half3, mask3, table2)
    return out.transpose(2, 0, 1)


# speedup vs baseline: 1.9148x; 1.1043x over previous
import functools

import jax
import jax.numpy as jnp
from jax import lax
from jax.experimental import pallas as pl
from jax.experimental.pallas import tpu as pltpu
from jax.experimental.pallas import tpu_sc as plsc

VOCAB = 1000000
DIM = 64
B = 4096
L = 50

NC = 2
NS = 16
LANES = 16
NW = NC * NS

ROWS = B * L
RPW = ROWS // NW
CHUNK = 128
NCHUNK = RPW // CHUNK
NGRP = CHUNK // LANES
OPC = CHUNK // 2

TBLK = 4096
NTBLK = (VOCAB + TBLK - 1) // TBLK
PREC = TBLK // 2
PTAB = NTBLK * PREC

_MAGIC = 0x5F3759DF


def _pack_body(tt_ref, out_ref):
    xt = tt_ref[...].T
    out_ref[...] = jnp.concatenate([xt[:PREC], xt[PREC:]], axis=1)


_pack_table = pl.pallas_call(
    _pack_body,
    grid=(NTBLK,),
    in_specs=[pl.BlockSpec((DIM, TBLK), lambda c: (0, c))],
    out_specs=pl.BlockSpec((PREC, DIM * 2), lambda c: (c, 0)),
    out_shape=jax.ShapeDtypeStruct((PTAB, DIM * 2), jnp.float32),
)


def _rsqrt(ssv):
    bits = plsc.bitcast(ssv, jnp.int32)
    y = plsc.bitcast(_MAGIC - (bits >> 1), jnp.float32)
    for _ in range(3):
        y = y * (1.5 - 0.5 * (ssv * y) * y)
    return y


_mesh = plsc.VectorSubcoreMesh(core_axis_name="c", subcore_axis_name="s")


@functools.partial(
    pl.kernel,
    mesh=_mesh,
    out_type=jax.ShapeDtypeStruct((L, DIM, B), jnp.float32),
    scratch_types=[
        pltpu.VMEM((NCHUNK, CHUNK), jnp.int32),
        pltpu.VMEM((NCHUNK, CHUNK), jnp.int32),
        pltpu.VMEM((NCHUNK, CHUNK), jnp.int32),
        pltpu.VMEM((CHUNK, DIM * 2), jnp.float32),
        pltpu.VMEM((CHUNK, DIM * 2), jnp.float32),
        pltpu.VMEM((DIM, CHUNK), jnp.float32),
        pltpu.VMEM((DIM, CHUNK), jnp.float32),
        pltpu.SemaphoreType.DMA,
        pltpu.SemaphoreType.DMA,
        pltpu.SemaphoreType.DMA,
        pltpu.SemaphoreType.DMA,
    ],
    compiler_params=pltpu.CompilerParams(needs_layout_passes=False),
)
def _sc_lookup(idx_hbm, half_hbm, mask_hbm, table_hbm, out_hbm,
               idx_v, half_v, msk_v, buf0, buf1, obuf0, obuf1,
               gsem0, gsem1, osem0, osem1):
    wid = lax.axis_index("s") * NC + lax.axis_index("c")
    col0 = wid * CHUNK

    pltpu.sync_copy(idx_hbm.at[wid], idx_v)
    pltpu.sync_copy(half_hbm.at[wid], half_v)
    pltpu.sync_copy(mask_hbm.at[wid], msk_v)

    def compute(j, buf, obuf):
        @plsc.parallel_loop(0, NGRP, 1, unroll=2)
        def grp_body(g):
            iota = lax.iota(jnp.int32, LANES)
            rows = g * LANES + iota
            h = half_v[j, pl.ds(g * LANES, LANES)]
            hcol = h * DIM
            acc = [None] * 4
            for k in range(DIM):
                v = plsc.load_gather(buf, [rows, hcol + k])
                a = acc[k & 3]
                acc[k & 3] = v * v if a is None else a + v * v
            ss = (acc[0] + acc[1]) + (acc[2] + acc[3])
            y = _rsqrt(ss)
            norm = ss * y
            mrow = msk_v[j, pl.ds(g * LANES, LANES)]
            bm = jnp.where(mrow > 0, 0.0, 1.0)
            inv = jnp.where(norm >= 1e-6, y, 1e6) * bm
            for k in range(DIM):
                v = plsc.load_gather(buf, [rows, hcol + k])
                obuf[k, pl.ds(g * LANES, LANES)] = v * inv

    def gather(j, buf, sem):
        return pltpu.async_copy(table_hbm.at[idx_v.at[j]], buf, sem)

    def writeback(j, obuf, sem):
        return pltpu.async_copy(obuf, out_hbm.at[j, :, pl.ds(col0, CHUNK)], sem)

    gather(0, buf0, gsem0)
    gather(1, buf1, gsem1)

    def stage(t, j, buf, obuf, gsem, osem):
        pltpu.make_async_copy(table_hbm.at[idx_v.at[j]], buf, gsem).wait()

        @pl.when(t > 0)
        def _():
            pltpu.make_async_copy(
                obuf, out_hbm.at[j, :, pl.ds(col0, CHUNK)], osem
            ).wait()

        compute(j, buf, obuf)
        writeback(j, obuf, osem)
        gather(jnp.minimum(j + 2, NCHUNK - 1), buf, gsem)

    def outer(t, carry):
        stage(t, 2 * t, buf0, obuf0, gsem0, osem0)
        stage(t, 2 * t + 1, buf1, obuf1, gsem1, osem1)
        return carry

    lax.fori_loop(0, NCHUNK // 2, outer, 0)

    pltpu.make_async_copy(table_hbm.at[idx_v.at[0]], buf0, gsem0).wait()
    pltpu.make_async_copy(table_hbm.at[idx_v.at[0]], buf1, gsem1).wait()
    pltpu.make_async_copy(obuf0, out_hbm.at[0, :, pl.ds(col0, CHUNK)], osem0).wait()
    pltpu.make_async_copy(obuf1, out_hbm.at[0, :, pl.ds(col0, CHUNK)], osem1).wait()


def kernel(indices, mask, table):
    slab = indices.reshape(NW, CHUNK, L).transpose(0, 2, 1)
    blk = slab >> 12
    q = slab & (TBLK - 1)
    half3 = q >> 11
    idx3 = (blk << 11) | (q & (PREC - 1))
    mask3 = mask.astype(jnp.int32).reshape(NW, CHUNK, L).transpose(0, 2, 1)
    table2 = _pack_table(table.T)
    out = _sc_lookup(idx3, half3, mask3, table2)
    return out.transpose(2, 0, 1)
